# baseline (speedup 1.0000x reference)
import jax
import jax.numpy as jnp
from jax.experimental import pallas as pl
from jax.experimental.pallas import tpu as pltpu

S = 2048
DIM = 2048
H = 16
DH = 128
BLOCK = 128
NB = S // BLOCK
WIN = 2
EPS = 1e-6
SCALE = 1.0 / (DH ** 0.5)

STRIP = 256
NSTRIP = S // STRIP
WBLK = STRIP // BLOCK + 2 * WIN
WK = WBLK * BLOCK

NT = 16
TILE_N = DIM // NT
NA = 3 * NT
MSPLIT = 4


def _fused_kernel(x_ref, wq_ref, wk_ref, wv_ref, gq_ref, gk_ref, wo_ref,
                  o_ref, qkv_ref):
    n = pl.program_id(0)

    def _norm(acc, g):
        segs = []
        for j in range(TILE_N // DH):
            seg = acc[:, j * DH:(j + 1) * DH]
            var = jnp.mean(seg * seg, axis=1, keepdims=True)
            segs.append(seg * jax.lax.rsqrt(var + EPS))
        gfull = jnp.concatenate([g] * (TILE_N // DH))
        return (jnp.concatenate(segs, axis=1) * gfull[None, :]).astype(jnp.bfloat16)

    def _proj(w_ref, g, col0):
        w = w_ref[...].astype(jnp.bfloat16)
        t = (n % NT) * TILE_N
        for i in range(MSPLIT):
            rows = x_ref[i * (S // MSPLIT):(i + 1) * (S // MSPLIT), :]
            acc = jnp.dot(rows, w, preferred_element_type=jnp.float32)
            res = _norm(acc, g) if g is not None else acc.astype(jnp.bfloat16)
            qkv_ref[pl.ds(i * (S // MSPLIT), S // MSPLIT),
                    pl.ds(col0 + t, TILE_N)] = res

    @pl.when(n < NT)
    def _():
        _proj(wq_ref, gq_ref[...], 0)

    @pl.when((n >= NT) & (n < 2 * NT))
    def _():
        _proj(wk_ref, gk_ref[...], DIM)

    @pl.when((n >= 2 * NT) & (n < NA))
    def _():
        _proj(wv_ref, None, 2 * DIM)

    @pl.when(n >= NA)
    def _():
        m_ = n - NA
        sidx = m_ // 2
        grp = m_ % 2
        qb0 = sidx * (STRIP // BLOCK)
        start_blk = jnp.clip(qb0 - WIN, 0, NB - WBLK)
        start = start_blk * BLOCK

        r = jax.lax.broadcasted_iota(jnp.int32, (STRIP, WK), 0)
        c = jax.lax.broadcasted_iota(jnp.int32, (STRIP, WK), 1)
        qb = qb0 + r // BLOCK
        jb = start_blk + c // BLOCK
        neg = jnp.where(jnp.abs(jb - qb) <= WIN,
                        jnp.float32(0), jnp.float32(-1e9))

        hbase = grp * (H // 2) * DH
        outs = []
        for h in range(H // 2):
            lo = h * DH
            qh = qkv_ref[pl.ds(sidx * STRIP, STRIP),
                         pl.ds(hbase + lo, DH)]
            kh = qkv_ref[pl.ds(start, WK), pl.ds(DIM + hbase + lo, DH)]
            vh = qkv_ref[pl.ds(start, WK), pl.ds(2 * DIM + hbase + lo, DH)]
            s = jax.lax.dot_general(
                qh, kh, (((1,), (1,)), ((), ())),
                preferred_element_type=jnp.float32) * SCALE + neg
            m = jnp.max(s, axis=1, keepdims=True)
            p = jnp.exp(s - m)
            l = jnp.sum(p, axis=1, keepdims=True)
            oh = jnp.dot(p.astype(jnp.bfloat16), vh,
                         preferred_element_type=jnp.float32)
            outs.append((oh / l).astype(jnp.bfloat16))

        a = jnp.concatenate(outs, axis=1)
        partial = jax.lax.dot_general(
            a, wo_ref[pl.ds(hbase, (H // 2) * DH), :],
            (((1,), (0,)), ((), ())),
            preferred_element_type=jnp.float32)

        @pl.when(grp == 0)
        def _():
            o_ref[...] = partial

        @pl.when(grp == 1)
        def _():
            o_ref[...] += partial


@jax.jit
def _run(x, Wq, Wk, Wv, Wo, gq, gk):
    out = pl.pallas_call(
        _fused_kernel,
        grid=(NA + 2 * NSTRIP,),
        in_specs=[
            pl.BlockSpec((S, DIM), lambda n: (0, 0)),
            pl.BlockSpec((DIM, TILE_N),
                         lambda n: (0, jnp.clip(n, 0, NT - 1))),
            pl.BlockSpec((DIM, TILE_N),
                         lambda n: (0, jnp.clip(n - NT, 0, NT - 1))),
            pl.BlockSpec((DIM, TILE_N),
                         lambda n: (0, jnp.clip(n - 2 * NT, 0, NT - 1))),
            pl.BlockSpec((DH,), lambda n: (0,)),
            pl.BlockSpec((DH,), lambda n: (0,)),
            pl.BlockSpec((DIM, DIM), lambda n: (0, 0)),
        ],
        out_specs=pl.BlockSpec(
            (STRIP, DIM), lambda n: (jnp.maximum(n - NA, 0) // 2, 0)),
        out_shape=jax.ShapeDtypeStruct((S, DIM), jnp.float32),
        scratch_shapes=[pltpu.VMEM((S, 3 * DIM), jnp.bfloat16)],
    )(x.astype(jnp.bfloat16), Wq, Wk, Wv, gq, gk, Wo.astype(jnp.bfloat16))

    return out


def kernel(x, Wq, Wk, Wv, Wo, gq, gk):
    return _run(x[0], Wq, Wk, Wv, Wo, gq, gk)[None]


# speedup vs baseline: 1.2592x; 1.2592x over previous
import jax
import jax.numpy as jnp
from jax.experimental import pallas as pl
from jax.experimental.pallas import tpu as pltpu

S = 2048
DIM = 2048
H = 16
DH = 128
BLOCK = 128
NB = S // BLOCK
WIN = 2
EPS = 1e-6
SCALE = 1.0 / (DH ** 0.5)

STRIP = 256
NSTRIP = S // STRIP
WBLK = STRIP // BLOCK + 2 * WIN
WK = WBLK * BLOCK

NT = 4
TILE_N = DIM // NT


def _qkv_kernel(x_ref, wq_ref, wk_ref, wv_ref, gq_ref, gk_ref, o_ref,
                xs_ref):
    n = pl.program_id(0)

    @pl.when(n == 0)
    def _():
        xs_ref[...] = x_ref[...].astype(jnp.bfloat16)

    def _norm(acc, g):
        segs = []
        for j in range(TILE_N // DH):
            seg = acc[:, j * DH:(j + 1) * DH]
            var = jnp.mean(seg * seg, axis=1, keepdims=True)
            segs.append(seg * jax.lax.rsqrt(var + EPS))
        gfull = jnp.concatenate([g] * (TILE_N // DH))
        return (jnp.concatenate(segs, axis=1) * gfull[None, :]).astype(jnp.bfloat16)

    MSPLIT = 2

    def _mm(w_ref, g):
        w = w_ref[...].astype(jnp.bfloat16)
        parts = []
        for i in range(MSPLIT):
            rows = xs_ref[i * (S // MSPLIT):(i + 1) * (S // MSPLIT), :]
            acc = jnp.dot(rows, w, preferred_element_type=jnp.float32)
            parts.append(_norm(acc, g) if g is not None
                         else acc.astype(jnp.bfloat16))
        return jnp.concatenate(parts, axis=0)

    @pl.when(n < NT)
    def _():
        o_ref[...] = _mm(wq_ref, gq_ref[...])

    @pl.when((n >= NT) & (n < 2 * NT))
    def _():
        o_ref[...] = _mm(wk_ref, gk_ref[...])

    @pl.when(n >= 2 * NT)
    def _():
        o_ref[...] = _mm(wv_ref, None)


def _attn_kernel(q_ref, k_ref, v_ref, wo_ref, o_ref):
    sidx = pl.program_id(0)
    qb0 = sidx * (STRIP // BLOCK)
    start_blk = jnp.clip(qb0 - WIN, 0, NB - WBLK)
    start = start_blk * BLOCK

    r = jax.lax.broadcasted_iota(jnp.int32, (STRIP, WK), 0)
    c = jax.lax.broadcasted_iota(jnp.int32, (STRIP, WK), 1)
    qb = qb0 + r // BLOCK
    jb = start_blk + c // BLOCK
    neg = jnp.where(jnp.abs(jb - qb) <= WIN,
                    jnp.float32(0), jnp.float32(-1e9))

    outs = []
    for h in range(H):
        lo, hi = h * DH, (h + 1) * DH
        qh = q_ref[:, lo:hi]
        kh = k_ref[pl.ds(start, WK), lo:hi]
        vh = v_ref[pl.ds(start, WK), lo:hi]
        s = jax.lax.dot_general(
            qh, kh, (((1,), (1,)), ((), ())),
            preferred_element_type=jnp.float32) * SCALE + neg
        m = jnp.max(s, axis=1, keepdims=True)
        p = jnp.exp(s - m)
        l = jnp.sum(p, axis=1, keepdims=True)
        oh = jnp.dot(p.astype(jnp.bfloat16), vh,
                     preferred_element_type=jnp.float32)
        outs.append((oh / l).astype(jnp.bfloat16))

    a = jnp.concatenate(outs, axis=1)
    o_ref[...] = jnp.dot(a, wo_ref[...], preferred_element_type=jnp.float32)


@jax.jit
def _run(x, Wq, Wk, Wv, Wo, gq, gk):
    qkv = pl.pallas_call(
        _qkv_kernel,
        grid=(3 * NT,),
        in_specs=[
            pl.BlockSpec((S, DIM), lambda n: (0, 0)),
            pl.BlockSpec((DIM, TILE_N),
                         lambda n: (0, jnp.clip(n, 0, NT - 1))),
            pl.BlockSpec((DIM, TILE_N),
                         lambda n: (0, jnp.clip(n - NT, 0, NT - 1))),
            pl.BlockSpec((DIM, TILE_N),
                         lambda n: (0, jnp.clip(n - 2 * NT, 0, NT - 1))),
            pl.BlockSpec((DH,), lambda n: (0,)),
            pl.BlockSpec((DH,), lambda n: (0,)),
        ],
        out_specs=pl.BlockSpec((S, TILE_N), lambda n: (0, n)),
        out_shape=jax.ShapeDtypeStruct((S, 3 * DIM), jnp.bfloat16),
        scratch_shapes=[pltpu.VMEM((S, DIM), jnp.bfloat16)],
    )(x, Wq, Wk, Wv, gq, gk)

    qn = qkv[:, :DIM]
    kn = qkv[:, DIM:2 * DIM]
    vv = qkv[:, 2 * DIM:]

    out = pl.pallas_call(
        _attn_kernel,
        grid=(NSTRIP,),
        in_specs=[
            pl.BlockSpec((STRIP, DIM), lambda s: (s, 0)),
            pl.BlockSpec((S, DIM), lambda s: (0, 0)),
            pl.BlockSpec((S, DIM), lambda s: (0, 0)),
            pl.BlockSpec((DIM, DIM), lambda s: (0, 0)),
        ],
        out_specs=pl.BlockSpec((STRIP, DIM), lambda s: (s, 0)),
        out_shape=jax.ShapeDtypeStruct((S, DIM), jnp.float32),
    )(qn, kn, vv, Wo.astype(jnp.bfloat16))

    return out


def kernel(x, Wq, Wk, Wv, Wo, gq, gk):
    return _run(x[0], Wq, Wk, Wv, Wo, gq, gk)[None]


# speedup vs baseline: 1.2669x; 1.0061x over previous
import jax
import jax.numpy as jnp
from jax.experimental import pallas as pl
from jax.experimental.pallas import tpu as pltpu

S = 2048
DIM = 2048
H = 16
DH = 128
BLOCK = 128
NB = S // BLOCK
WIN = 2
EPS = 1e-6
SCALE = 1.0 / (DH ** 0.5)

STRIP = 256
NSTRIP = S // STRIP
WBLK = STRIP // BLOCK + 2 * WIN
WK = WBLK * BLOCK

NT = 4
TILE_N = DIM // NT


def _qkv_kernel(x_ref, wq_ref, wk_ref, wv_ref, gq_ref, gk_ref, o_ref,
                xs_ref):
    n = pl.program_id(0)

    @pl.when(n == 0)
    def _():
        xs_ref[...] = x_ref[...].astype(jnp.bfloat16)

    def _norm(acc, g):
        segs = []
        for j in range(TILE_N // DH):
            seg = acc[:, j * DH:(j + 1) * DH]
            var = jnp.mean(seg * seg, axis=1, keepdims=True)
            segs.append(seg * jax.lax.rsqrt(var + EPS))
        gfull = jnp.concatenate([g] * (TILE_N // DH))
        return (jnp.concatenate(segs, axis=1) * gfull[None, :]).astype(jnp.bfloat16)

    MSPLIT = 8

    def _mm(w_ref, g):
        w = w_ref[...].astype(jnp.bfloat16)
        parts = []
        for i in range(MSPLIT):
            rows = xs_ref[i * (S // MSPLIT):(i + 1) * (S // MSPLIT), :]
            acc = jnp.dot(rows, w, preferred_element_type=jnp.float32)
            parts.append(_norm(acc, g) if g is not None
                         else acc.astype(jnp.bfloat16))
        return jnp.concatenate(parts, axis=0)

    @pl.when(n < NT)
    def _():
        o_ref[...] = _mm(wq_ref, gq_ref[...])

    @pl.when((n >= NT) & (n < 2 * NT))
    def _():
        o_ref[...] = _mm(wk_ref, gk_ref[...])

    @pl.when(n >= 2 * NT)
    def _():
        o_ref[...] = _mm(wv_ref, None)


def _attn_kernel(q_ref, k_ref, v_ref, wo_ref, o_ref):
    sidx = pl.program_id(0)
    qb0 = sidx * (STRIP // BLOCK)
    start_blk = jnp.clip(qb0 - WIN, 0, NB - WBLK)
    start = start_blk * BLOCK

    r = jax.lax.broadcasted_iota(jnp.int32, (STRIP, WK), 0)
    c = jax.lax.broadcasted_iota(jnp.int32, (STRIP, WK), 1)
    qb = qb0 + r // BLOCK
    jb = start_blk + c // BLOCK
    neg = jnp.where(jnp.abs(jb - qb) <= WIN,
                    jnp.float32(0), jnp.float32(-1e9))

    outs = []
    for h in range(H):
        lo, hi = h * DH, (h + 1) * DH
        qh = q_ref[:, lo:hi]
        kh = k_ref[pl.ds(start, WK), lo:hi]
        vh = v_ref[pl.ds(start, WK), lo:hi]
        s = jax.lax.dot_general(
            qh, kh, (((1,), (1,)), ((), ())),
            preferred_element_type=jnp.float32) * SCALE + neg
        m = jnp.max(s, axis=1, keepdims=True)
        p = jnp.exp(s - m)
        l = jnp.sum(p, axis=1, keepdims=True)
        oh = jnp.dot(p.astype(jnp.bfloat16), vh,
                     preferred_element_type=jnp.float32)
        outs.append((oh / l).astype(jnp.bfloat16))

    a = jnp.concatenate(outs, axis=1)
    o_ref[...] = jnp.dot(a, wo_ref[...], preferred_element_type=jnp.float32)


@jax.jit
def _run(x, Wq, Wk, Wv, Wo, gq, gk):
    qkv = pl.pallas_call(
        _qkv_kernel,
        grid=(3 * NT,),
        in_specs=[
            pl.BlockSpec((S, DIM), lambda n: (0, 0)),
            pl.BlockSpec((DIM, TILE_N),
                         lambda n: (0, jnp.clip(n, 0, NT - 1))),
            pl.BlockSpec((DIM, TILE_N),
                         lambda n: (0, jnp.clip(n - NT, 0, NT - 1))),
            pl.BlockSpec((DIM, TILE_N),
                         lambda n: (0, jnp.clip(n - 2 * NT, 0, NT - 1))),
            pl.BlockSpec((DH,), lambda n: (0,)),
            pl.BlockSpec((DH,), lambda n: (0,)),
        ],
        out_specs=pl.BlockSpec((S, TILE_N), lambda n: (0, n)),
        out_shape=jax.ShapeDtypeStruct((S, 3 * DIM), jnp.bfloat16),
        scratch_shapes=[pltpu.VMEM((S, DIM), jnp.bfloat16)],
    )(x, Wq, Wk, Wv, gq, gk)

    qn = qkv[:, :DIM]
    kn = qkv[:, DIM:2 * DIM]
    vv = qkv[:, 2 * DIM:]

    out = pl.pallas_call(
        _attn_kernel,
        grid=(NSTRIP,),
        in_specs=[
            pl.BlockSpec((STRIP, DIM), lambda s: (s, 0)),
            pl.BlockSpec((S, DIM), lambda s: (0, 0)),
            pl.BlockSpec((S, DIM), lambda s: (0, 0)),
            pl.BlockSpec((DIM, DIM), lambda s: (0, 0)),
        ],
        out_specs=pl.BlockSpec((STRIP, DIM), lambda s: (s, 0)),
        out_shape=jax.ShapeDtypeStruct((S, DIM), jnp.float32),
    )(qn, kn, vv, Wo.astype(jnp.bfloat16))

    return out


def kernel(x, Wq, Wk, Wv, Wo, gq, gk):
    return _run(x[0], Wq, Wk, Wv, Wo, gq, gk)[None]
